# baseline (speedup 1.0000x reference)
import functools

import jax
import jax.numpy as jnp
from jax import lax
from jax.experimental import pallas as pl
from jax.experimental.pallas import tpu as pltpu
from jax.experimental.pallas import tpu_sc as plsc

B, C, H, W, D = 2, 192, 32, 32, 16
NH, NP = 8, 4
HD = C // NH
N = H * W * D
T = B * N
NQ = NH * NP * 8
TM = 512


def _frontend_body(x_ref, g_ref, bg_ref, wqT_ref, bq_ref, w1T_ref, b1_ref,
                   w2x_ref, b2x_ref, w2y_ref, b2y_ref, w2z_ref, b2z_ref,
                   waT_ref, ba_ref, msk_ref, dup_ref,
                   idxlo_ref, idxhi_ref, wgtlo_ref, wgthi_ref):
    f32 = jnp.float32
    Xb = x_ref[0]
    m = jnp.mean(Xb, axis=0, keepdims=True)
    xc = Xb - m
    v = jnp.mean(xc * xc, axis=0, keepdims=True)
    Xn = xc * lax.rsqrt(v + 1e-5) * g_ref[...] + bg_ref[...]
    dnT = (((0,), (0,)), ((), ()))
    Q = lax.dot_general(Xn, wqT_ref[...], dnT, preferred_element_type=f32) + bq_ref[...]
    Hd = jnp.maximum(jnp.dot(Q, w1T_ref[...], preferred_element_type=f32) + b1_ref[...], 0.0)
    offx = jnp.clip(jnp.dot(Hd, w2x_ref[...], preferred_element_type=f32) + b2x_ref[...], -3.0, 3.0)
    offy = jnp.clip(jnp.dot(Hd, w2y_ref[...], preferred_element_type=f32) + b2y_ref[...], -3.0, 3.0)
    offz = jnp.clip(jnp.dot(Hd, w2z_ref[...], preferred_element_type=f32) + b2z_ref[...], -3.0, 3.0)
    A = jnp.dot(Q, waT_ref[...], preferred_element_type=f32) + ba_ref[...]
    A = A - jnp.max(A, axis=1, keepdims=True)
    E = jnp.exp(A)
    den = jnp.dot(E, msk_ref[...], preferred_element_type=f32)
    awq = jnp.dot(E / den, dup_ref[...], preferred_element_type=f32)

    bidx = pl.program_id(0)
    n = pl.program_id(1) * TM + lax.broadcasted_iota(jnp.int32, (TM, 1), 0)
    gh = n // (W * D)
    gw = (n // D) % W
    gd = n % D
    x = jnp.clip(gh.astype(f32) + offx, 0.0, W - 1.0)
    y = jnp.clip(gw.astype(f32) + offy, 0.0, H - 1.0)
    z = jnp.clip(gd.astype(f32) + offz, 0.0, D - 1.0)
    x0f = jnp.floor(x); y0f = jnp.floor(y); z0f = jnp.floor(z)
    wx = x - x0f; wy = y - y0f; wz = z - z0f
    qi = lax.broadcasted_iota(jnp.int32, (TM, NQ), 1)
    c8 = qi >> 5
    cxb = (c8 & 1) > 0
    cyb = (c8 & 2) > 0
    czb = (c8 & 4) > 0
    xi = jnp.minimum(x0f.astype(jnp.int32) + (c8 & 1), W - 1)
    yi = jnp.minimum(y0f.astype(jnp.int32) + ((c8 >> 1) & 1), H - 1)
    zi = jnp.minimum(z0f.astype(jnp.int32) + ((c8 >> 2) & 1), D - 1)
    wxc = jnp.where(cxb, wx, 1.0 - wx)
    wyc = jnp.where(cyb, wy, 1.0 - wy)
    wzc = jnp.where(czb, wz, 1.0 - wz)
    hq = (qi & 31) >> 2
    gidx = ((bidx * NH + hq) << 14) + (yi * W + xi) * D + zi
    wq = awq * wxc * wyc * wzc
    idxlo_ref[...] = gidx[:, :128]
    idxhi_ref[...] = gidx[:, 128:]
    wgtlo_ref[...] = wq[:, :128]
    wgthi_ref[...] = wq[:, 128:]


NW = 32
TPW = T // NW
KT = 8
NCH = TPW // KT
ROWS = KT * NQ
IR = ROWS // 128
HDP = 32
CP = NH * HDP


def _sc_gather_body(g_ref, idxlo_ref, idxhi_ref, wgtlo_ref, wgthi_ref,
                    outlo_ref, outhi_ref, idxb, wgtb, rowb, outb, sem0, sem1):
    sems = (sem0, sem1)
    wid = lax.axis_index("s") * 2 + lax.axis_index("c")
    tok0 = wid * TPW

    def fill(slot, ch):
        t0 = tok0 + ch * KT
        pltpu.sync_copy(idxlo_ref.at[pl.ds(t0, KT)], idxb.at[slot, pl.ds(0, KT)])
        pltpu.sync_copy(idxhi_ref.at[pl.ds(t0, KT)], idxb.at[slot, pl.ds(KT, KT)])
        pltpu.sync_copy(wgtlo_ref.at[pl.ds(t0, KT)], wgtb.at[slot, pl.ds(0, KT)])
        pltpu.sync_copy(wgthi_ref.at[pl.ds(t0, KT)], wgtb.at[slot, pl.ds(KT, KT)])
        for j in range(IR):
            pltpu.async_copy(g_ref.at[idxb.at[slot, j]],
                             rowb.at[slot, pl.ds(j * 128, 128)], sems[slot])

    def drain(slot):
        pltpu.make_async_copy(g_ref.at[pl.ds(0, ROWS)], rowb.at[slot],
                              sems[slot]).wait()

    def compute(slot, ch):
        def token_body(t, _):
            for h in range(NH):
                acc_e = jnp.zeros((16,), jnp.float32)
                acc_o = jnp.zeros((16,), jnp.float32)
                for c8 in range(8):
                    wrow = t + KT * (c8 // 4)
                    wcol = (c8 % 4) * 32 + (h // 4) * 16
                    wv16 = wgtb[slot, wrow, pl.ds(wcol, 16)]
                    for p in range(NP):
                        q = c8 * 32 + h * NP + p
                        wv = jnp.full((16,), wv16[(h % 4) * NP + p], jnp.float32)
                        r = (q // 128) * (KT * 128) + t * 128 + (q % 128)
                        v = rowb[slot, r, pl.ds(0, HDP // 2)]
                        ve = plsc.bitcast(jnp.left_shift(v, 16), jnp.float32)
                        vo = plsc.bitcast(jnp.bitwise_and(v, jnp.int32(-65536)), jnp.float32)
                        acc_e = acc_e + wv * ve
                        acc_o = acc_o + wv * vo
                orow = t + KT * (h // 4)
                ocol = (h % 4) * 32
                outb[slot, orow, pl.ds(ocol, 16)] = acc_e
                outb[slot, orow, pl.ds(ocol + 16, 16)] = acc_o
            return 0
        lax.fori_loop(0, KT, token_body, 0)
        t0 = tok0 + ch * KT
        pltpu.sync_copy(outb.at[slot, pl.ds(0, KT)], outlo_ref.at[pl.ds(t0, KT)])
        pltpu.sync_copy(outb.at[slot, pl.ds(KT, KT)], outhi_ref.at[pl.ds(t0, KT)])

    fill(0, 0)
    fill(1, 1)

    def chunk_body(i, _):
        g = i * 2
        for slot in range(2):
            ch = g + slot
            drain(slot)
            compute(slot, ch)

            @pl.when(ch + 2 < NCH)
            def _():
                fill(slot, ch + 2)
        return 0

    lax.fori_loop(0, NCH // 2, chunk_body, 0)


@functools.partial(
    pl.kernel,
    out_type=[jax.ShapeDtypeStruct((T, 128), jnp.float32),
              jax.ShapeDtypeStruct((T, 128), jnp.float32)],
    mesh=plsc.VectorSubcoreMesh(core_axis_name="c", subcore_axis_name="s"),
    compiler_params=pltpu.CompilerParams(use_tc_tiling_on_sc=False,
                                         needs_layout_passes=False),
    scratch_types=[
        pltpu.VMEM((2, 2 * KT, 128), jnp.int32),
        pltpu.VMEM((2, 2 * KT, 128), jnp.float32),
        pltpu.VMEM((2, ROWS, HDP // 2), jnp.int32),
        pltpu.VMEM((2, 2 * KT, 128), jnp.float32),
        pltpu.SemaphoreType.DMA,
        pltpu.SemaphoreType.DMA,
    ],
)
def _sc_gather(g_ref, idxlo_ref, idxhi_ref, wgtlo_ref, wgthi_ref,
               outlo_ref, outhi_ref, idxb, wgtb, rowb, outb, sem0, sem1):
    _sc_gather_body(g_ref, idxlo_ref, idxhi_ref, wgtlo_ref, wgthi_ref,
                    outlo_ref, outhi_ref, idxb, wgtb, rowb, outb, sem0, sem1)


def _proj_body(olo_ref, ohi_ref, wlo_ref, whi_ref, bo_ref, out_ref):
    f32 = jnp.float32
    dn = (((1,), (1,)), ((), ()))
    out_ref[0] = (lax.dot_general(wlo_ref[...], olo_ref[...], dn, preferred_element_type=f32)
                  + lax.dot_general(whi_ref[...], ohi_ref[...], dn, preferred_element_type=f32)
                  + bo_ref[...])


def kernel(f_query, f_kv, ln_q_g, ln_q_b, ln_kv_g, ln_kv_b, Wq, bq, W1, b1, W2, b2, Wa, ba, Wo, bo):
    f32 = jnp.float32
    X3 = f_query.reshape(B, C, N)
    jq = jnp.arange(NQ) % 32
    W2T3 = W2.T.reshape(C, NH * NP, 3)
    w2x = W2T3[:, jq, 0]; w2y = W2T3[:, jq, 1]; w2z = W2T3[:, jq, 2]
    b23 = b2.reshape(NH * NP, 3)
    b2x = b23[jq, 0][None, :]; b2y = b23[jq, 1][None, :]; b2z = b23[jq, 2][None, :]
    jj = jnp.arange(NH * NP)
    msk = (jj[:, None] // NP == jj[None, :] // NP).astype(f32)
    dup = (jj[:, None] == (jnp.arange(NQ)[None, :] % 32)).astype(f32)

    col = lambda v: v.reshape(-1, 1)
    full = lambda s: pl.BlockSpec(s, lambda b, i: tuple(0 for _ in s))
    idxlo, idxhi, wgtlo, wgthi = pl.pallas_call(
        _frontend_body,
        grid=(B, N // TM),
        in_specs=[
            pl.BlockSpec((1, C, TM), lambda b, i: (b, 0, i)),
            full((C, 1)), full((C, 1)),
            full((C, C)), full((1, C)),
            full((C, C)), full((1, C)),
            full((C, NQ)), full((1, NQ)),
            full((C, NQ)), full((1, NQ)),
            full((C, NQ)), full((1, NQ)),
            full((C, NH * NP)), full((1, NH * NP)),
            full((NH * NP, NH * NP)), full((NH * NP, NQ)),
        ],
        out_specs=[pl.BlockSpec((TM, 128), lambda b, i: (b * (N // TM) + i, 0))] * 4,
        out_shape=[jax.ShapeDtypeStruct((T, 128), jnp.int32),
                   jax.ShapeDtypeStruct((T, 128), jnp.int32),
                   jax.ShapeDtypeStruct((T, 128), f32),
                   jax.ShapeDtypeStruct((T, 128), f32)],
    )(X3, col(ln_q_g), col(ln_q_b), Wq.T, bq[None, :], W1.T, b1[None, :],
      w2x, b2x, w2y, b2y, w2z, b2z, Wa.T, ba[None, :], msk, dup)

    G = lax.bitcast_convert_type(
        jnp.pad(
            f_kv.astype(jnp.bfloat16).reshape(B, NH, HD, H, W, D).transpose(0, 1, 3, 4, 5, 2),
            ((0, 0), (0, 0), (0, 0), (0, 0), (0, 0), (0, HDP - HD)),
        ).reshape(B * NH * N, HDP // 2, 2),
        jnp.int32)

    O_lo, O_hi = _sc_gather(G, idxlo, idxhi, wgtlo, wgthi)

    perm = jnp.concatenate([jnp.arange(0, HDP, 2), jnp.arange(1, HDP, 2)])
    WoP = jnp.take(
        jnp.pad(Wo.T.reshape(NH, HD, C), ((0, 0), (0, HDP - HD), (0, 0))),
        perm, axis=1).reshape(CP, C)
    w_lo = WoP[:128].T
    w_hi = WoP[128:].T

    out = pl.pallas_call(
        _proj_body,
        grid=(B, N // TM),
        in_specs=[
            pl.BlockSpec((TM, 128), lambda b, i: (b * (N // TM) + i, 0)),
            pl.BlockSpec((TM, 128), lambda b, i: (b * (N // TM) + i, 0)),
            full((C, 128)), full((C, 128)), full((C, 1)),
        ],
        out_specs=pl.BlockSpec((1, C, TM), lambda b, i: (b, 0, i)),
        out_shape=jax.ShapeDtypeStruct((B, C, N), f32),
    )(O_lo, O_hi, w_lo, w_hi, col(bo))

    return out.reshape(B, C, H, W, D)


# speedup vs baseline: 1.4517x; 1.4517x over previous
import functools

import jax
import jax.numpy as jnp
from jax import lax
from jax.experimental import pallas as pl
from jax.experimental.pallas import tpu as pltpu
from jax.experimental.pallas import tpu_sc as plsc

B, C, H, W, D = 2, 192, 32, 32, 16
NH, NP = 8, 4
HD = C // NH
N = H * W * D
T = B * N
NQ = NH * NP * 8
TM = 512


def _frontend_body(x_ref, g_ref, bg_ref, wqT_ref, bq_ref, w1T_ref, b1_ref,
                   w2x_ref, b2x_ref, w2y_ref, b2y_ref, w2z_ref, b2z_ref,
                   waT_ref, ba_ref, msk_ref, dup_ref,
                   idxlo_ref, idxhi_ref, wgtlo_ref, wgthi_ref):
    f32 = jnp.float32
    Xb = x_ref[0]
    m = jnp.mean(Xb, axis=0, keepdims=True)
    xc = Xb - m
    v = jnp.mean(xc * xc, axis=0, keepdims=True)
    Xn = xc * lax.rsqrt(v + 1e-5) * g_ref[...] + bg_ref[...]
    dnT = (((0,), (0,)), ((), ()))
    Q = lax.dot_general(Xn, wqT_ref[...], dnT, preferred_element_type=f32) + bq_ref[...]
    Hd = jnp.maximum(jnp.dot(Q, w1T_ref[...], preferred_element_type=f32) + b1_ref[...], 0.0)
    offx = jnp.clip(jnp.dot(Hd, w2x_ref[...], preferred_element_type=f32) + b2x_ref[...], -3.0, 3.0)
    offy = jnp.clip(jnp.dot(Hd, w2y_ref[...], preferred_element_type=f32) + b2y_ref[...], -3.0, 3.0)
    offz = jnp.clip(jnp.dot(Hd, w2z_ref[...], preferred_element_type=f32) + b2z_ref[...], -3.0, 3.0)
    A = jnp.dot(Q, waT_ref[...], preferred_element_type=f32) + ba_ref[...]
    A = A - jnp.max(A, axis=1, keepdims=True)
    E = jnp.exp(A)
    den = jnp.dot(E, msk_ref[...], preferred_element_type=f32)
    awq = jnp.dot(E / den, dup_ref[...], preferred_element_type=f32)

    bidx = pl.program_id(0)
    n = pl.program_id(1) * TM + lax.broadcasted_iota(jnp.int32, (TM, 1), 0)
    gh = n // (W * D)
    gw = (n // D) % W
    gd = n % D
    x = jnp.clip(gh.astype(f32) + offx, 0.0, W - 1.0)
    y = jnp.clip(gw.astype(f32) + offy, 0.0, H - 1.0)
    z = jnp.clip(gd.astype(f32) + offz, 0.0, D - 1.0)
    x0f = jnp.floor(x); y0f = jnp.floor(y); z0f = jnp.floor(z)
    wx = x - x0f; wy = y - y0f; wz = z - z0f
    qi = lax.broadcasted_iota(jnp.int32, (TM, NQ), 1)
    c8 = qi >> 5
    cxb = (c8 & 1) > 0
    cyb = (c8 & 2) > 0
    czb = (c8 & 4) > 0
    xi = jnp.minimum(x0f.astype(jnp.int32) + (c8 & 1), W - 1)
    yi = jnp.minimum(y0f.astype(jnp.int32) + ((c8 >> 1) & 1), H - 1)
    zi = jnp.minimum(z0f.astype(jnp.int32) + ((c8 >> 2) & 1), D - 1)
    wxc = jnp.where(cxb, wx, 1.0 - wx)
    wyc = jnp.where(cyb, wy, 1.0 - wy)
    wzc = jnp.where(czb, wz, 1.0 - wz)
    hq = (qi & 31) >> 2
    gidx = ((bidx * NH + hq) << 14) + (yi * W + xi) * D + zi
    wq = awq * wxc * wyc * wzc
    idxlo_ref[...] = gidx[:, :128]
    idxhi_ref[...] = gidx[:, 128:]
    wgtlo_ref[...] = wq[:, :128]
    wgthi_ref[...] = wq[:, 128:]


NW = 32
TPW = T // NW
KT = 8
NCH = TPW // KT
ROWS = KT * NQ
IR = ROWS // 128
HDP = 32
CP = NH * HDP


def _sc_gather_body(g_ref, idxlo_ref, idxhi_ref, wgtlo_ref, wgthi_ref,
                    outlo_ref, outhi_ref, idxb, wgtb, rowb, outb,
                    m0, m1, m2, m3, g0, g1, o0, o1):
    msems = (m0, m1, m2, m3)
    gsems = (g0, g1)
    osems = (o0, o1)
    wid = lax.axis_index("s") * 2 + lax.axis_index("c")
    tok0 = wid * TPW

    def stage(ms, ch):
        t0 = tok0 + ch * KT
        pltpu.async_copy(idxlo_ref.at[pl.ds(t0, KT)], idxb.at[ms, pl.ds(0, KT)], msems[ms])
        pltpu.async_copy(idxhi_ref.at[pl.ds(t0, KT)], idxb.at[ms, pl.ds(KT, KT)], msems[ms])
        pltpu.async_copy(wgtlo_ref.at[pl.ds(t0, KT)], wgtb.at[ms, pl.ds(0, KT)], msems[ms])
        pltpu.async_copy(wgthi_ref.at[pl.ds(t0, KT)], wgtb.at[ms, pl.ds(KT, KT)], msems[ms])

    def wait_meta(ms):
        pltpu.make_async_copy(idxlo_ref.at[pl.ds(0, 2 * KT)], idxb.at[ms], msems[ms]).wait()
        pltpu.make_async_copy(wgtlo_ref.at[pl.ds(0, 2 * KT)], wgtb.at[ms], msems[ms]).wait()

    def fire(ms, gs):
        for j in range(IR):
            pltpu.async_copy(g_ref.at[idxb.at[ms, j]],
                             rowb.at[gs, pl.ds(j * 128, 128)], gsems[gs])

    def drain_gather(gs):
        pltpu.make_async_copy(g_ref.at[pl.ds(0, ROWS)], rowb.at[gs], gsems[gs]).wait()

    def drain_out(os, t0):
        pltpu.make_async_copy(outb.at[os, pl.ds(0, KT)], outlo_ref.at[pl.ds(t0, KT)],
                              osems[os]).wait()
        pltpu.make_async_copy(outb.at[os, pl.ds(KT, KT)], outhi_ref.at[pl.ds(t0, KT)],
                              osems[os]).wait()

    def compute(ms, gs, os):
        def token_body(t, _):
            for h in range(NH):
                acc_e = jnp.zeros((16,), jnp.float32)
                acc_o = jnp.zeros((16,), jnp.float32)
                for c8 in range(8):
                    wrow = t + KT * (c8 // 4)
                    wcol = (c8 % 4) * 32 + (h // 4) * 16
                    wv16 = wgtb[ms, wrow, pl.ds(wcol, 16)]
                    for p in range(NP):
                        q = c8 * 32 + h * NP + p
                        wv = jnp.full((16,), wv16[(h % 4) * NP + p], jnp.float32)
                        r = (q // 128) * (KT * 128) + t * 128 + (q % 128)
                        v = rowb[gs, r, pl.ds(0, HDP // 2)]
                        ve = plsc.bitcast(jnp.left_shift(v, 16), jnp.float32)
                        vo = plsc.bitcast(jnp.bitwise_and(v, jnp.int32(-65536)), jnp.float32)
                        acc_e = acc_e + wv * ve
                        acc_o = acc_o + wv * vo
                orow = t + KT * (h // 4)
                ocol = (h % 4) * 32
                outb[os, orow, pl.ds(ocol, 16)] = acc_e
                outb[os, orow, pl.ds(ocol + 16, 16)] = acc_o
            return 0
        lax.fori_loop(0, KT, token_body, 0)

    def put_out(os, ch):
        t0 = tok0 + ch * KT
        pltpu.async_copy(outb.at[os, pl.ds(0, KT)], outlo_ref.at[pl.ds(t0, KT)], osems[os])
        pltpu.async_copy(outb.at[os, pl.ds(KT, KT)], outhi_ref.at[pl.ds(t0, KT)], osems[os])

    stage(0, 0)
    stage(1, 1)
    stage(2, 2)
    wait_meta(0)
    fire(0, 0)

    def quad_body(i, _):
        for k in range(4):
            ch = i * 4 + k
            gs = k % 2
            os = k % 2
            drain_gather(gs)

            @pl.when(ch + 1 < NCH)
            def _():
                wait_meta((k + 1) % 4)
                fire((k + 1) % 4, (k + 1) % 2)

            @pl.when(ch >= 2)
            def _():
                drain_out(os, tok0 + (ch - 2) * KT)

            compute(k, gs, os)
            put_out(os, ch)

            @pl.when(ch + 3 < NCH)
            def _():
                stage((k + 3) % 4, ch + 3)
        return 0

    lax.fori_loop(0, NCH // 4, quad_body, 0)
    drain_out(0, tok0 + (NCH - 2) * KT)
    drain_out(1, tok0 + (NCH - 1) * KT)


@functools.partial(
    pl.kernel,
    out_type=[jax.ShapeDtypeStruct((T, 128), jnp.float32),
              jax.ShapeDtypeStruct((T, 128), jnp.float32)],
    mesh=plsc.VectorSubcoreMesh(core_axis_name="c", subcore_axis_name="s"),
    compiler_params=pltpu.CompilerParams(use_tc_tiling_on_sc=False,
                                         needs_layout_passes=False),
    scratch_types=[
        pltpu.VMEM((4, 2 * KT, 128), jnp.int32),
        pltpu.VMEM((4, 2 * KT, 128), jnp.float32),
        pltpu.VMEM((2, ROWS, HDP // 2), jnp.int32),
        pltpu.VMEM((2, 2 * KT, 128), jnp.float32),
    ] + [pltpu.SemaphoreType.DMA] * 8,
)
def _sc_gather(g_ref, idxlo_ref, idxhi_ref, wgtlo_ref, wgthi_ref,
               outlo_ref, outhi_ref, idxb, wgtb, rowb, outb, *sems):
    _sc_gather_body(g_ref, idxlo_ref, idxhi_ref, wgtlo_ref, wgthi_ref,
                    outlo_ref, outhi_ref, idxb, wgtb, rowb, outb, *sems)


def _proj_body(olo_ref, ohi_ref, wlo_ref, whi_ref, bo_ref, out_ref):
    f32 = jnp.float32
    dn = (((1,), (1,)), ((), ()))
    out_ref[0] = (lax.dot_general(wlo_ref[...], olo_ref[...], dn, preferred_element_type=f32)
                  + lax.dot_general(whi_ref[...], ohi_ref[...], dn, preferred_element_type=f32)
                  + bo_ref[...])


def kernel(f_query, f_kv, ln_q_g, ln_q_b, ln_kv_g, ln_kv_b, Wq, bq, W1, b1, W2, b2, Wa, ba, Wo, bo):
    f32 = jnp.float32
    X3 = f_query.reshape(B, C, N)
    jq = jnp.arange(NQ) % 32
    W2T3 = W2.T.reshape(C, NH * NP, 3)
    w2x = W2T3[:, jq, 0]; w2y = W2T3[:, jq, 1]; w2z = W2T3[:, jq, 2]
    b23 = b2.reshape(NH * NP, 3)
    b2x = b23[jq, 0][None, :]; b2y = b23[jq, 1][None, :]; b2z = b23[jq, 2][None, :]
    jj = jnp.arange(NH * NP)
    msk = (jj[:, None] // NP == jj[None, :] // NP).astype(f32)
    dup = (jj[:, None] == (jnp.arange(NQ)[None, :] % 32)).astype(f32)

    col = lambda v: v.reshape(-1, 1)
    full = lambda s: pl.BlockSpec(s, lambda b, i: tuple(0 for _ in s))
    idxlo, idxhi, wgtlo, wgthi = pl.pallas_call(
        _frontend_body,
        grid=(B, N // TM),
        in_specs=[
            pl.BlockSpec((1, C, TM), lambda b, i: (b, 0, i)),
            full((C, 1)), full((C, 1)),
            full((C, C)), full((1, C)),
            full((C, C)), full((1, C)),
            full((C, NQ)), full((1, NQ)),
            full((C, NQ)), full((1, NQ)),
            full((C, NQ)), full((1, NQ)),
            full((C, NH * NP)), full((1, NH * NP)),
            full((NH * NP, NH * NP)), full((NH * NP, NQ)),
        ],
        out_specs=[pl.BlockSpec((TM, 128), lambda b, i: (b * (N // TM) + i, 0))] * 4,
        out_shape=[jax.ShapeDtypeStruct((T, 128), jnp.int32),
                   jax.ShapeDtypeStruct((T, 128), jnp.int32),
                   jax.ShapeDtypeStruct((T, 128), f32),
                   jax.ShapeDtypeStruct((T, 128), f32)],
    )(X3, col(ln_q_g), col(ln_q_b), Wq.T, bq[None, :], W1.T, b1[None, :],
      w2x, b2x, w2y, b2y, w2z, b2z, Wa.T, ba[None, :], msk, dup)

    G = lax.bitcast_convert_type(
        jnp.pad(
            f_kv.reshape(B, NH, HD, N).transpose(0, 1, 3, 2).astype(jnp.bfloat16),
            ((0, 0), (0, 0), (0, 0), (0, HDP - HD)),
        ).reshape(B * NH * N, HDP // 2, 2),
        jnp.int32)

    O_lo, O_hi = _sc_gather(G, idxlo, idxhi, wgtlo, wgthi)

    perm = jnp.concatenate([jnp.arange(0, HDP, 2), jnp.arange(1, HDP, 2)])
    WoP = jnp.take(
        jnp.pad(Wo.T.reshape(NH, HD, C), ((0, 0), (0, HDP - HD), (0, 0))),
        perm, axis=1).reshape(CP, C)
    w_lo = WoP[:128].T
    w_hi = WoP[128:].T

    out = pl.pallas_call(
        _proj_body,
        grid=(B, N // TM),
        in_specs=[
            pl.BlockSpec((TM, 128), lambda b, i: (b * (N // TM) + i, 0)),
            pl.BlockSpec((TM, 128), lambda b, i: (b * (N // TM) + i, 0)),
            full((C, 128)), full((C, 128)), full((C, 1)),
        ],
        out_specs=pl.BlockSpec((1, C, TM), lambda b, i: (b, 0, i)),
        out_shape=jax.ShapeDtypeStruct((B, C, N), f32),
    )(O_lo, O_hi, w_lo, w_hi, col(bo))

    return out.reshape(B, C, H, W, D)


# speedup vs baseline: 1.9546x; 1.3465x over previous
import functools

import jax
import jax.numpy as jnp
from jax import lax
from jax.experimental import pallas as pl
from jax.experimental.pallas import tpu as pltpu
from jax.experimental.pallas import tpu_sc as plsc

B, C, H, W, D = 2, 192, 32, 32, 16
NH, NP = 8, 4
HD = C // NH
N = H * W * D
T = B * N
NQ = NH * NP * 8
TM = 512


def _frontend_body(x_ref, g_ref, bg_ref, wqT_ref, bq_ref, w1T_ref, b1_ref,
                   w2x_ref, b2x_ref, w2y_ref, b2y_ref, w2z_ref, b2z_ref,
                   waT_ref, ba_ref, msk_ref, dup_ref,
                   idxlo_ref, idxhi_ref, wgtlo_ref, wgthi_ref):
    f32 = jnp.float32
    Xb = x_ref[...]
    m = jnp.mean(Xb, axis=1, keepdims=True)
    xc = Xb - m
    v = jnp.mean(xc * xc, axis=1, keepdims=True)
    Xn = xc * lax.rsqrt(v + 1e-5) * g_ref[...] + bg_ref[...]
    Q = jnp.dot(Xn, wqT_ref[...], preferred_element_type=f32) + bq_ref[...]
    Hd = jnp.maximum(jnp.dot(Q, w1T_ref[...], preferred_element_type=f32) + b1_ref[...], 0.0)
    offx = jnp.clip(jnp.dot(Hd, w2x_ref[...], preferred_element_type=f32) + b2x_ref[...], -3.0, 3.0)
    offy = jnp.clip(jnp.dot(Hd, w2y_ref[...], preferred_element_type=f32) + b2y_ref[...], -3.0, 3.0)
    offz = jnp.clip(jnp.dot(Hd, w2z_ref[...], preferred_element_type=f32) + b2z_ref[...], -3.0, 3.0)
    A = jnp.dot(Q, waT_ref[...], preferred_element_type=f32) + ba_ref[...]
    A = A - jnp.max(A, axis=1, keepdims=True)
    E = jnp.exp(A)
    den = jnp.dot(E, msk_ref[...], preferred_element_type=f32)
    awq = jnp.dot(E / den, dup_ref[...], preferred_element_type=f32)

    bidx = pl.program_id(0)
    n = pl.program_id(1) * TM + lax.broadcasted_iota(jnp.int32, (TM, 1), 0)
    gh = n // (W * D)
    gw = (n // D) % W
    gd = n % D
    x = jnp.clip(gh.astype(f32) + offx, 0.0, W - 1.0)
    y = jnp.clip(gw.astype(f32) + offy, 0.0, H - 1.0)
    z = jnp.clip(gd.astype(f32) + offz, 0.0, D - 1.0)
    x0f = jnp.floor(x); y0f = jnp.floor(y); z0f = jnp.floor(z)
    wx = x - x0f; wy = y - y0f; wz = z - z0f
    qi = lax.broadcasted_iota(jnp.int32, (TM, NQ), 1)
    c8 = qi >> 5
    cxb = (c8 & 1) > 0
    cyb = (c8 & 2) > 0
    czb = (c8 & 4) > 0
    xi = jnp.minimum(x0f.astype(jnp.int32) + (c8 & 1), W - 1)
    yi = jnp.minimum(y0f.astype(jnp.int32) + ((c8 >> 1) & 1), H - 1)
    zi = jnp.minimum(z0f.astype(jnp.int32) + ((c8 >> 2) & 1), D - 1)
    wxc = jnp.where(cxb, wx, 1.0 - wx)
    wyc = jnp.where(cyb, wy, 1.0 - wy)
    wzc = jnp.where(czb, wz, 1.0 - wz)
    hq = (qi & 31) >> 2
    gidx = ((bidx << 14) + (yi * W + xi) * D + zi) * NH + hq
    wq = awq * wxc * wyc * wzc
    idxlo_ref[...] = gidx[:, :128]
    idxhi_ref[...] = gidx[:, 128:]
    wgtlo_ref[...] = wq[:, :128]
    wgthi_ref[...] = wq[:, 128:]


NW = 32
TPW = T // NW
KT = 8
NCH = TPW // KT
ROWS = KT * NQ
IR = ROWS // 128
HDP = 32
CP = NH * HDP


def _sc_gather_body(g_ref, idxlo_ref, idxhi_ref, wgtlo_ref, wgthi_ref,
                    outlo_ref, outhi_ref, idxb, wgtb, rowb, outb,
                    m0, m1, m2, m3, g0, g1, o0, o1):
    msems = (m0, m1, m2, m3)
    gsems = (g0, g1)
    osems = (o0, o1)
    wid = lax.axis_index("s") * 2 + lax.axis_index("c")
    tok0 = wid * TPW

    def stage(ms, ch):
        t0 = tok0 + ch * KT
        pltpu.async_copy(idxlo_ref.at[pl.ds(t0, KT)], idxb.at[ms, pl.ds(0, KT)], msems[ms])
        pltpu.async_copy(idxhi_ref.at[pl.ds(t0, KT)], idxb.at[ms, pl.ds(KT, KT)], msems[ms])
        pltpu.async_copy(wgtlo_ref.at[pl.ds(t0, KT)], wgtb.at[ms, pl.ds(0, KT)], msems[ms])
        pltpu.async_copy(wgthi_ref.at[pl.ds(t0, KT)], wgtb.at[ms, pl.ds(KT, KT)], msems[ms])

    def wait_meta(ms):
        pltpu.make_async_copy(idxlo_ref.at[pl.ds(0, 2 * KT)], idxb.at[ms], msems[ms]).wait()
        pltpu.make_async_copy(wgtlo_ref.at[pl.ds(0, 2 * KT)], wgtb.at[ms], msems[ms]).wait()

    def fire(ms, gs):
        for j in range(IR):
            pltpu.async_copy(g_ref.at[idxb.at[ms, j]],
                             rowb.at[gs, pl.ds(j * 128, 128)], gsems[gs])

    def drain_gather(gs):
        pltpu.make_async_copy(g_ref.at[pl.ds(0, ROWS)], rowb.at[gs], gsems[gs]).wait()

    def drain_out(os, t0):
        pltpu.make_async_copy(outb.at[os, pl.ds(0, KT)], outlo_ref.at[pl.ds(t0, KT)],
                              osems[os]).wait()
        pltpu.make_async_copy(outb.at[os, pl.ds(KT, KT)], outhi_ref.at[pl.ds(t0, KT)],
                              osems[os]).wait()

    def compute(ms, gs, os):
        def token_body(t, _):
            for h in range(NH):
                acc_e = jnp.zeros((16,), jnp.float32)
                acc_o = jnp.zeros((16,), jnp.float32)
                for c8 in range(8):
                    wrow = t + KT * (c8 // 4)
                    wcol = (c8 % 4) * 32 + (h // 4) * 16
                    wv16 = wgtb[ms, wrow, pl.ds(wcol, 16)]
                    for p in range(NP):
                        q = c8 * 32 + h * NP + p
                        wv = jnp.full((16,), wv16[(h % 4) * NP + p], jnp.float32)
                        r = (q // 128) * (KT * 128) + t * 128 + (q % 128)
                        v = plsc.bitcast(rowb[gs, r, pl.ds(0, HDP)], jnp.int32)
                        ve = plsc.bitcast(jnp.left_shift(v, 16), jnp.float32)
                        vo = plsc.bitcast(jnp.bitwise_and(v, jnp.int32(-65536)), jnp.float32)
                        acc_e = acc_e + wv * ve
                        acc_o = acc_o + wv * vo
                orow = t + KT * (h // 4)
                ocol = (h % 4) * 32
                outb[os, orow, pl.ds(ocol, 16)] = acc_e
                outb[os, orow, pl.ds(ocol + 16, 16)] = acc_o
            return 0
        lax.fori_loop(0, KT, token_body, 0)

    def put_out(os, ch):
        t0 = tok0 + ch * KT
        pltpu.async_copy(outb.at[os, pl.ds(0, KT)], outlo_ref.at[pl.ds(t0, KT)], osems[os])
        pltpu.async_copy(outb.at[os, pl.ds(KT, KT)], outhi_ref.at[pl.ds(t0, KT)], osems[os])

    stage(0, 0)
    stage(1, 1)
    stage(2, 2)
    wait_meta(0)
    fire(0, 0)

    def quad_body(i, _):
        for k in range(4):
            ch = i * 4 + k
            gs = k % 2
            os = k % 2
            drain_gather(gs)

            @pl.when(ch + 1 < NCH)
            def _():
                wait_meta((k + 1) % 4)
                fire((k + 1) % 4, (k + 1) % 2)

            @pl.when(ch >= 2)
            def _():
                drain_out(os, tok0 + (ch - 2) * KT)

            compute(k, gs, os)
            put_out(os, ch)

            @pl.when(ch + 3 < NCH)
            def _():
                stage((k + 3) % 4, ch + 3)
        return 0

    lax.fori_loop(0, NCH // 4, quad_body, 0)
    drain_out(0, tok0 + (NCH - 2) * KT)
    drain_out(1, tok0 + (NCH - 1) * KT)


@functools.partial(
    pl.kernel,
    out_type=[jax.ShapeDtypeStruct((T, 128), jnp.float32),
              jax.ShapeDtypeStruct((T, 128), jnp.float32)],
    mesh=plsc.VectorSubcoreMesh(core_axis_name="c", subcore_axis_name="s"),
    compiler_params=pltpu.CompilerParams(use_tc_tiling_on_sc=False,
                                         needs_layout_passes=False),
    scratch_types=[
        pltpu.VMEM((4, 2 * KT, 128), jnp.int32),
        pltpu.VMEM((4, 2 * KT, 128), jnp.float32),
        pltpu.VMEM((2, ROWS, HDP), jnp.bfloat16),
        pltpu.VMEM((2, 2 * KT, 128), jnp.float32),
    ] + [pltpu.SemaphoreType.DMA] * 8,
)
def _sc_gather(g_ref, idxlo_ref, idxhi_ref, wgtlo_ref, wgthi_ref,
               outlo_ref, outhi_ref, idxb, wgtb, rowb, outb, *sems):
    _sc_gather_body(g_ref, idxlo_ref, idxhi_ref, wgtlo_ref, wgthi_ref,
                    outlo_ref, outhi_ref, idxb, wgtb, rowb, outb, *sems)


def _proj_body(olo_ref, ohi_ref, wlo_ref, whi_ref, bo_ref, out_ref):
    f32 = jnp.float32
    out_ref[...] = (jnp.dot(olo_ref[...], wlo_ref[...], preferred_element_type=f32)
                    + jnp.dot(ohi_ref[...], whi_ref[...], preferred_element_type=f32)
                    + bo_ref[...])


def kernel(f_query, f_kv, ln_q_g, ln_q_b, ln_kv_g, ln_kv_b, Wq, bq, W1, b1, W2, b2, Wa, ba, Wo, bo):
    f32 = jnp.float32
    X = f_query.reshape(B, C, N).transpose(0, 2, 1).reshape(T, C)
    jq = jnp.arange(NQ) % 32
    W2T3 = W2.T.reshape(C, NH * NP, 3)
    w2x = W2T3[:, jq, 0]; w2y = W2T3[:, jq, 1]; w2z = W2T3[:, jq, 2]
    b23 = b2.reshape(NH * NP, 3)
    b2x = b23[jq, 0][None, :]; b2y = b23[jq, 1][None, :]; b2z = b23[jq, 2][None, :]
    jj = jnp.arange(NH * NP)
    msk = (jj[:, None] // NP == jj[None, :] // NP).astype(f32)
    dup = (jj[:, None] == (jnp.arange(NQ)[None, :] % 32)).astype(f32)

    row = lambda v: v.reshape(1, -1)
    full = lambda s: pl.BlockSpec(s, lambda b, i: tuple(0 for _ in s))
    idxlo, idxhi, wgtlo, wgthi = pl.pallas_call(
        _frontend_body,
        grid=(B, N // TM),
        in_specs=[
            pl.BlockSpec((TM, C), lambda b, i: (b * (N // TM) + i, 0)),
            full((1, C)), full((1, C)),
            full((C, C)), full((1, C)),
            full((C, C)), full((1, C)),
            full((C, NQ)), full((1, NQ)),
            full((C, NQ)), full((1, NQ)),
            full((C, NQ)), full((1, NQ)),
            full((C, NH * NP)), full((1, NH * NP)),
            full((NH * NP, NH * NP)), full((NH * NP, NQ)),
        ],
        out_specs=[pl.BlockSpec((TM, 128), lambda b, i: (b * (N // TM) + i, 0))] * 4,
        out_shape=[jax.ShapeDtypeStruct((T, 128), jnp.int32),
                   jax.ShapeDtypeStruct((T, 128), jnp.int32),
                   jax.ShapeDtypeStruct((T, 128), f32),
                   jax.ShapeDtypeStruct((T, 128), f32)],
    )(X, row(ln_q_g), row(ln_q_b), Wq.T, bq[None, :], W1.T, b1[None, :],
      w2x, b2x, w2y, b2y, w2z, b2z, Wa.T, ba[None, :], msk, dup)

    G = jnp.pad(
        f_kv.reshape(B, C, N).transpose(0, 2, 1).reshape(B, N, NH, HD),
        ((0, 0), (0, 0), (0, 0), (0, HDP - HD)),
    ).astype(jnp.bfloat16).reshape(B * N * NH, HDP)

    O_lo, O_hi = _sc_gather(G, idxlo, idxhi, wgtlo, wgthi)

    perm = jnp.concatenate([jnp.arange(0, HDP, 2), jnp.arange(1, HDP, 2)])
    WoP = jnp.take(
        jnp.pad(Wo.T.reshape(NH, HD, C), ((0, 0), (0, HDP - HD), (0, 0))),
        perm, axis=1).reshape(CP, C)
    w_lo = WoP[:128]
    w_hi = WoP[128:]

    out = pl.pallas_call(
        _proj_body,
        grid=(B, N // TM),
        in_specs=[
            pl.BlockSpec((TM, 128), lambda b, i: (b * (N // TM) + i, 0)),
            pl.BlockSpec((TM, 128), lambda b, i: (b * (N // TM) + i, 0)),
            full((128, C)), full((128, C)), full((1, C)),
        ],
        out_specs=pl.BlockSpec((TM, C), lambda b, i: (b * (N // TM) + i, 0)),
        out_shape=jax.ShapeDtypeStruct((T, C), f32),
    )(O_lo, O_hi, w_lo, w_hi, row(bo))

    return out.reshape(B, N, C).transpose(0, 2, 1).reshape(B, C, H, W, D)


# speedup vs baseline: 2.0348x; 1.0410x over previous
import functools

import jax
import jax.numpy as jnp
from jax import lax
from jax.experimental import pallas as pl
from jax.experimental.pallas import tpu as pltpu
from jax.experimental.pallas import tpu_sc as plsc

B, C, H, W, D = 2, 192, 32, 32, 16
NH, NP = 8, 4
HD = C // NH
N = H * W * D
T = B * N
NQ = NH * NP * 8
TM = 512


def _frontend_body(x_ref, g_ref, bg_ref, wqT_ref, bq_ref, w1T_ref, b1_ref,
                   w2x_ref, b2x_ref, w2y_ref, b2y_ref, w2z_ref, b2z_ref,
                   waT_ref, ba_ref, msk_ref, dup_ref,
                   idxlo_ref, idxhi_ref, wgtlo_ref, wgthi_ref):
    f32 = jnp.float32
    Xb = x_ref[...]
    m = jnp.mean(Xb, axis=1, keepdims=True)
    xc = Xb - m
    v = jnp.mean(xc * xc, axis=1, keepdims=True)
    Xn = xc * lax.rsqrt(v + 1e-5) * g_ref[...] + bg_ref[...]
    Q = jnp.dot(Xn, wqT_ref[...], preferred_element_type=f32) + bq_ref[...]
    Hd = jnp.maximum(jnp.dot(Q, w1T_ref[...], preferred_element_type=f32) + b1_ref[...], 0.0)
    offx = jnp.clip(jnp.dot(Hd, w2x_ref[...], preferred_element_type=f32) + b2x_ref[...], -3.0, 3.0)
    offy = jnp.clip(jnp.dot(Hd, w2y_ref[...], preferred_element_type=f32) + b2y_ref[...], -3.0, 3.0)
    offz = jnp.clip(jnp.dot(Hd, w2z_ref[...], preferred_element_type=f32) + b2z_ref[...], -3.0, 3.0)
    A = jnp.dot(Q, waT_ref[...], preferred_element_type=f32) + ba_ref[...]
    A = A - jnp.max(A, axis=1, keepdims=True)
    E = jnp.exp(A)
    den = jnp.dot(E, msk_ref[...], preferred_element_type=f32)
    awq = jnp.dot(E / den, dup_ref[...], preferred_element_type=f32)

    bidx = pl.program_id(0)
    n = pl.program_id(1) * TM + lax.broadcasted_iota(jnp.int32, (TM, 1), 0)
    gh = n // (W * D)
    gw = (n // D) % W
    gd = n % D
    x = jnp.clip(gh.astype(f32) + offx, 0.0, W - 1.0)
    y = jnp.clip(gw.astype(f32) + offy, 0.0, H - 1.0)
    z = jnp.clip(gd.astype(f32) + offz, 0.0, D - 1.0)
    x0f = jnp.floor(x); y0f = jnp.floor(y); z0f = jnp.floor(z)
    wx = x - x0f; wy = y - y0f; wz = z - z0f
    qi = lax.broadcasted_iota(jnp.int32, (TM, NQ), 1)
    c8 = qi >> 5
    cxb = (c8 & 1) > 0
    cyb = (c8 & 2) > 0
    czb = (c8 & 4) > 0
    xi = jnp.minimum(x0f.astype(jnp.int32) + (c8 & 1), W - 1)
    yi = jnp.minimum(y0f.astype(jnp.int32) + ((c8 >> 1) & 1), H - 1)
    zi = jnp.minimum(z0f.astype(jnp.int32) + ((c8 >> 2) & 1), D - 1)
    wxc = jnp.where(cxb, wx, 1.0 - wx)
    wyc = jnp.where(cyb, wy, 1.0 - wy)
    wzc = jnp.where(czb, wz, 1.0 - wz)
    hq = (qi & 31) >> 2
    gidx = ((bidx << 14) + (yi * W + xi) * D + zi) * NH + hq
    wq = awq * wxc * wyc * wzc
    idxlo_ref[...] = gidx[:, :128]
    idxhi_ref[...] = gidx[:, 128:]
    wgtlo_ref[...] = wq[:, :128]
    wgthi_ref[...] = wq[:, 128:]


NW = 32
TPW = T // NW
KT = 8
NCH = TPW // KT
ROWS = KT * NQ
IR = ROWS // 128
HDP = 32
CP = NH * HDP


def _sc_gather_body(g_ref, idxlo_ref, idxhi_ref, wgtlo_ref, wgthi_ref,
                    outlo_ref, outhi_ref, idxb, wgtb, rowb, outb,
                    m0, m1, m2, m3, g0, g1, o0, o1):
    msems = (m0, m1, m2, m3)
    gsems = (g0, g1)
    osems = (o0, o1)
    g2 = g_ref
    wid = lax.axis_index("s") * 2 + lax.axis_index("c")
    tok0 = wid * TPW

    def stage(ms, ch):
        t0 = tok0 + ch * KT
        pltpu.async_copy(idxlo_ref.at[pl.ds(t0, KT)], idxb.at[ms, pl.ds(0, KT)], msems[ms])
        pltpu.async_copy(idxhi_ref.at[pl.ds(t0, KT)], idxb.at[ms, pl.ds(KT, KT)], msems[ms])
        pltpu.async_copy(wgtlo_ref.at[pl.ds(t0, KT)], wgtb.at[ms, pl.ds(0, KT)], msems[ms])
        pltpu.async_copy(wgthi_ref.at[pl.ds(t0, KT)], wgtb.at[ms, pl.ds(KT, KT)], msems[ms])

    def wait_meta(ms):
        pltpu.make_async_copy(idxlo_ref.at[pl.ds(0, 2 * KT)], idxb.at[ms], msems[ms]).wait()
        pltpu.make_async_copy(wgtlo_ref.at[pl.ds(0, 2 * KT)], wgtb.at[ms], msems[ms]).wait()

    def fire(ms, gs):
        for j in range(IR):
            pltpu.async_copy(g2.at[idxb.at[ms, j]],
                             rowb.at[gs, pl.ds(j * 128, 128)], gsems[gs])

    def drain_gather(gs):
        pltpu.make_async_copy(g2.at[pl.ds(0, ROWS)], rowb.at[gs], gsems[gs]).wait()

    def drain_out(os, t0):
        pltpu.make_async_copy(outb.at[os, pl.ds(0, KT)], outlo_ref.at[pl.ds(t0, KT)],
                              osems[os]).wait()
        pltpu.make_async_copy(outb.at[os, pl.ds(KT, KT)], outhi_ref.at[pl.ds(t0, KT)],
                              osems[os]).wait()

    def compute(ms, gs, os):
        def token_body(t, _):
            for h in range(NH):
                acc_e = jnp.zeros((16,), jnp.float32)
                acc_o = jnp.zeros((16,), jnp.float32)
                for c8 in range(8):
                    wrow = t + KT * (c8 // 4)
                    wcol = (c8 % 4) * 32 + (h // 4) * 16
                    wv16 = wgtb[ms, wrow, pl.ds(wcol, 16)]
                    for p in range(NP):
                        q = c8 * 32 + h * NP + p
                        wv = jnp.full((16,), wv16[(h % 4) * NP + p], jnp.float32)
                        r = (q // 128) * (KT * 128) + t * 128 + (q % 128)
                        v = plsc.bitcast(rowb[gs, r, pl.ds(0, HDP)], jnp.int32)
                        ve = plsc.bitcast(jnp.left_shift(v, 16), jnp.float32)
                        vo = plsc.bitcast(v, jnp.float32)
                        acc_e = acc_e + wv * ve
                        acc_o = acc_o + wv * vo
                orow = t + KT * (h // 4)
                ocol = (h % 4) * 32
                outb[os, orow, pl.ds(ocol, 16)] = acc_e
                outb[os, orow, pl.ds(ocol + 16, 16)] = acc_o
            return 0
        lax.fori_loop(0, KT, token_body, 0)

    def put_out(os, ch):
        t0 = tok0 + ch * KT
        pltpu.async_copy(outb.at[os, pl.ds(0, KT)], outlo_ref.at[pl.ds(t0, KT)], osems[os])
        pltpu.async_copy(outb.at[os, pl.ds(KT, KT)], outhi_ref.at[pl.ds(t0, KT)], osems[os])

    stage(0, 0)
    stage(1, 1)
    stage(2, 2)
    wait_meta(0)
    fire(0, 0)

    def quad_body(i, _):
        for k in range(4):
            ch = i * 4 + k
            gs = k % 2
            os = k % 2

            @pl.when(ch + 1 < NCH)
            def _():
                wait_meta((k + 1) % 4)
                fire((k + 1) % 4, (k + 1) % 2)

            drain_gather(gs)

            @pl.when(ch >= 2)
            def _():
                drain_out(os, tok0 + (ch - 2) * KT)

            compute(k, gs, os)
            put_out(os, ch)

            @pl.when(ch + 3 < NCH)
            def _():
                stage((k + 3) % 4, ch + 3)
        return 0

    lax.fori_loop(0, NCH // 4, quad_body, 0)
    drain_out(0, tok0 + (NCH - 2) * KT)
    drain_out(1, tok0 + (NCH - 1) * KT)


@functools.partial(
    pl.kernel,
    out_type=[jax.ShapeDtypeStruct((T, 128), jnp.float32),
              jax.ShapeDtypeStruct((T, 128), jnp.float32)],
    mesh=plsc.VectorSubcoreMesh(core_axis_name="c", subcore_axis_name="s"),
    compiler_params=pltpu.CompilerParams(use_tc_tiling_on_sc=False,
                                         needs_layout_passes=False),
    scratch_types=[
        pltpu.VMEM((4, 2 * KT, 128), jnp.int32),
        pltpu.VMEM((4, 2 * KT, 128), jnp.float32),
        pltpu.VMEM((2, ROWS, HDP), jnp.bfloat16),
        pltpu.VMEM((2, 2 * KT, 128), jnp.float32),
    ] + [pltpu.SemaphoreType.DMA] * 8,
)
def _sc_gather(g_ref, idxlo_ref, idxhi_ref, wgtlo_ref, wgthi_ref,
               outlo_ref, outhi_ref, idxb, wgtb, rowb, outb, *sems):
    _sc_gather_body(g_ref, idxlo_ref, idxhi_ref, wgtlo_ref, wgthi_ref,
                    outlo_ref, outhi_ref, idxb, wgtb, rowb, outb, *sems)


def _pack_body(x_ref, sele_ref, selo_ref, out_ref):
    f32 = jnp.float32
    xb = x_ref[...]
    ev = jnp.dot(xb, sele_ref[...], preferred_element_type=f32).astype(jnp.bfloat16)
    od = jnp.dot(xb, selo_ref[...], preferred_element_type=f32).astype(jnp.bfloat16)
    li = pltpu.bitcast(ev, jnp.uint16).astype(jnp.int32)
    hi = pltpu.bitcast(od, jnp.uint16).astype(jnp.int32) << 16
    out_ref[...] = li | hi


def _proj_body(olo_ref, ohi_ref, wlo_ref, whi_ref, bo_ref, out_ref):
    f32 = jnp.float32
    out_ref[...] = (jnp.dot(olo_ref[...], wlo_ref[...], preferred_element_type=f32)
                    + jnp.dot(ohi_ref[...], whi_ref[...], preferred_element_type=f32)
                    + bo_ref[...])


def kernel(f_query, f_kv, ln_q_g, ln_q_b, ln_kv_g, ln_kv_b, Wq, bq, W1, b1, W2, b2, Wa, ba, Wo, bo):
    f32 = jnp.float32
    X = f_query.reshape(B, C, N).transpose(0, 2, 1).reshape(T, C)
    jq = jnp.arange(NQ) % 32
    W2T3 = W2.T.reshape(C, NH * NP, 3)
    w2x = W2T3[:, jq, 0]; w2y = W2T3[:, jq, 1]; w2z = W2T3[:, jq, 2]
    b23 = b2.reshape(NH * NP, 3)
    b2x = b23[jq, 0][None, :]; b2y = b23[jq, 1][None, :]; b2z = b23[jq, 2][None, :]
    jj = jnp.arange(NH * NP)
    msk = (jj[:, None] // NP == jj[None, :] // NP).astype(f32)
    dup = (jj[:, None] == (jnp.arange(NQ)[None, :] % 32)).astype(f32)

    row = lambda v: v.reshape(1, -1)
    full = lambda s: pl.BlockSpec(s, lambda b, i: tuple(0 for _ in s))
    idxlo, idxhi, wgtlo, wgthi = pl.pallas_call(
        _frontend_body,
        grid=(B, N // TM),
        in_specs=[
            pl.BlockSpec((TM, C), lambda b, i: (b * (N // TM) + i, 0)),
            full((1, C)), full((1, C)),
            full((C, C)), full((1, C)),
            full((C, C)), full((1, C)),
            full((C, NQ)), full((1, NQ)),
            full((C, NQ)), full((1, NQ)),
            full((C, NQ)), full((1, NQ)),
            full((C, NH * NP)), full((1, NH * NP)),
            full((NH * NP, NH * NP)), full((NH * NP, NQ)),
        ],
        out_specs=[pl.BlockSpec((TM, 128), lambda b, i: (b * (N // TM) + i, 0))] * 4,
        out_shape=[jax.ShapeDtypeStruct((T, 128), jnp.int32),
                   jax.ShapeDtypeStruct((T, 128), jnp.int32),
                   jax.ShapeDtypeStruct((T, 128), f32),
                   jax.ShapeDtypeStruct((T, 128), f32)],
    )(X, row(ln_q_g), row(ln_q_b), Wq.T, bq[None, :], W1.T, b1[None, :],
      w2x, b2x, w2y, b2y, w2z, b2z, Wa.T, ba[None, :], msk, dup)

    G = jnp.pad(
        f_kv.reshape(B, C, N).transpose(0, 2, 1).reshape(B, N, NH, HD),
        ((0, 0), (0, 0), (0, 0), (0, HDP - HD)),
    ).astype(jnp.bfloat16).reshape(B * N * NH, HDP)

    O_lo, O_hi = _sc_gather(G, idxlo, idxhi, wgtlo, wgthi)

    perm = jnp.concatenate([jnp.arange(0, HDP, 2), jnp.arange(1, HDP, 2)])
    WoP = jnp.take(
        jnp.pad(Wo.T.reshape(NH, HD, C), ((0, 0), (0, HDP - HD), (0, 0))),
        perm, axis=1).reshape(CP, C)
    w_lo = WoP[:128]
    w_hi = WoP[128:]

    out = pl.pallas_call(
        _proj_body,
        grid=(B, N // TM),
        in_specs=[
            pl.BlockSpec((TM, 128), lambda b, i: (b * (N // TM) + i, 0)),
            pl.BlockSpec((TM, 128), lambda b, i: (b * (N // TM) + i, 0)),
            full((128, C)), full((128, C)), full((1, C)),
        ],
        out_specs=pl.BlockSpec((TM, C), lambda b, i: (b * (N // TM) + i, 0)),
        out_shape=jax.ShapeDtypeStruct((T, C), f32),
    )(O_lo, O_hi, w_lo, w_hi, row(bo))

    return out.reshape(B, N, C).transpose(0, 2, 1).reshape(B, C, H, W, D)


# speedup vs baseline: 2.1095x; 1.0367x over previous
import functools

import jax
import jax.numpy as jnp
from jax import lax
from jax.experimental import pallas as pl
from jax.experimental.pallas import tpu as pltpu
from jax.experimental.pallas import tpu_sc as plsc

B, C, H, W, D = 2, 192, 32, 32, 16
NH, NP = 8, 4
HD = C // NH
N = H * W * D
T = B * N
NQ = NH * NP * 8
TM = 1024


def _frontend_body(x_ref, g_ref, bg_ref, wqT_ref, bq_ref, w1T_ref, b1_ref,
                   w2x_ref, b2x_ref, w2y_ref, b2y_ref, w2z_ref, b2z_ref,
                   waT_ref, ba_ref, msk_ref, dup_ref,
                   idxlo_ref, idxhi_ref, wgtlo_ref, wgthi_ref):
    f32 = jnp.float32
    Xb = x_ref[...]
    m = jnp.mean(Xb, axis=1, keepdims=True)
    xc = Xb - m
    v = jnp.mean(xc * xc, axis=1, keepdims=True)
    Xn = xc * lax.rsqrt(v + 1e-5) * g_ref[...] + bg_ref[...]
    Q = jnp.dot(Xn, wqT_ref[...], preferred_element_type=f32) + bq_ref[...]
    Hd = jnp.maximum(jnp.dot(Q, w1T_ref[...], preferred_element_type=f32) + b1_ref[...], 0.0)
    offx = jnp.clip(jnp.dot(Hd, w2x_ref[...], preferred_element_type=f32) + b2x_ref[...], -3.0, 3.0)
    offy = jnp.clip(jnp.dot(Hd, w2y_ref[...], preferred_element_type=f32) + b2y_ref[...], -3.0, 3.0)
    offz = jnp.clip(jnp.dot(Hd, w2z_ref[...], preferred_element_type=f32) + b2z_ref[...], -3.0, 3.0)
    A = jnp.dot(Q, waT_ref[...], preferred_element_type=f32) + ba_ref[...]
    A = A - jnp.max(A, axis=1, keepdims=True)
    E = jnp.exp(A)
    den = jnp.dot(E, msk_ref[...], preferred_element_type=f32)
    awq = jnp.dot(E / den, dup_ref[...], preferred_element_type=f32)

    bidx = pl.program_id(0)
    n = pl.program_id(1) * TM + lax.broadcasted_iota(jnp.int32, (TM, 1), 0)
    gh = n // (W * D)
    gw = (n // D) % W
    gd = n % D
    x = jnp.clip(gh.astype(f32) + offx, 0.0, W - 1.0)
    y = jnp.clip(gw.astype(f32) + offy, 0.0, H - 1.0)
    z = jnp.clip(gd.astype(f32) + offz, 0.0, D - 1.0)
    x0f = jnp.floor(x); y0f = jnp.floor(y); z0f = jnp.floor(z)
    wx = x - x0f; wy = y - y0f; wz = z - z0f
    qi = lax.broadcasted_iota(jnp.int32, (TM, NQ), 1)
    c8 = qi >> 5
    cxb = (c8 & 1) > 0
    cyb = (c8 & 2) > 0
    czb = (c8 & 4) > 0
    xi = jnp.minimum(x0f.astype(jnp.int32) + (c8 & 1), W - 1)
    yi = jnp.minimum(y0f.astype(jnp.int32) + ((c8 >> 1) & 1), H - 1)
    zi = jnp.minimum(z0f.astype(jnp.int32) + ((c8 >> 2) & 1), D - 1)
    wxc = jnp.where(cxb, wx, 1.0 - wx)
    wyc = jnp.where(cyb, wy, 1.0 - wy)
    wzc = jnp.where(czb, wz, 1.0 - wz)
    hq = (qi & 31) >> 2
    gidx = ((bidx << 14) + (yi * W + xi) * D + zi) * NH + hq
    wq = awq * wxc * wyc * wzc
    idxlo_ref[...] = gidx[:, :128]
    idxhi_ref[...] = gidx[:, 128:]
    wgtlo_ref[...] = wq[:, :128]
    wgthi_ref[...] = wq[:, 128:]


NW = 32
TPW = T // NW
KT = 8
NCH = TPW // KT
ROWS = KT * NQ
IR = ROWS // 128
HDP = 32
CP = NH * HDP


def _sc_gather_body(g_ref, idxlo_ref, idxhi_ref, wgtlo_ref, wgthi_ref,
                    outlo_ref, outhi_ref, idxb, wgtb, rowb, outb,
                    m0, m1, m2, m3, g0, g1, o0, o1):
    msems = (m0, m1, m2, m3)
    gsems = (g0, g1)
    osems = (o0, o1)
    g2 = g_ref
    wid = lax.axis_index("s") * 2 + lax.axis_index("c")
    tok0 = wid * TPW

    def stage(ms, ch):
        t0 = tok0 + ch * KT
        pltpu.async_copy(idxlo_ref.at[pl.ds(t0, KT)], idxb.at[ms, pl.ds(0, KT)], msems[ms])
        pltpu.async_copy(idxhi_ref.at[pl.ds(t0, KT)], idxb.at[ms, pl.ds(KT, KT)], msems[ms])
        pltpu.async_copy(wgtlo_ref.at[pl.ds(t0, KT)], wgtb.at[ms, pl.ds(0, KT)], msems[ms])
        pltpu.async_copy(wgthi_ref.at[pl.ds(t0, KT)], wgtb.at[ms, pl.ds(KT, KT)], msems[ms])

    def wait_meta(ms):
        pltpu.make_async_copy(idxlo_ref.at[pl.ds(0, 2 * KT)], idxb.at[ms], msems[ms]).wait()
        pltpu.make_async_copy(wgtlo_ref.at[pl.ds(0, 2 * KT)], wgtb.at[ms], msems[ms]).wait()

    def fire(ms, gs):
        for j in range(IR):
            pltpu.async_copy(g2.at[idxb.at[ms, j]],
                             rowb.at[gs, pl.ds(j * 128, 128)], gsems[gs])

    def drain_gather(gs):
        pltpu.make_async_copy(g2.at[pl.ds(0, ROWS)], rowb.at[gs], gsems[gs]).wait()

    def drain_out(os, t0):
        pltpu.make_async_copy(outb.at[os, pl.ds(0, KT)], outlo_ref.at[pl.ds(t0, KT)],
                              osems[os]).wait()
        pltpu.make_async_copy(outb.at[os, pl.ds(KT, KT)], outhi_ref.at[pl.ds(t0, KT)],
                              osems[os]).wait()

    def compute(ms, gs, os):
        def token_body(t, _):
            for h in range(NH):
                acc_e = jnp.zeros((16,), jnp.float32)
                acc_o = jnp.zeros((16,), jnp.float32)
                for c8 in range(8):
                    wrow = t + KT * (c8 // 4)
                    wcol = (c8 % 4) * 32 + (h // 4) * 16
                    wv16 = wgtb[ms, wrow, pl.ds(wcol, 16)]
                    for p in range(NP):
                        q = c8 * 32 + h * NP + p
                        wv = jnp.full((16,), wv16[(h % 4) * NP + p], jnp.float32)
                        r = (q // 128) * (KT * 128) + t * 128 + (q % 128)
                        v = plsc.bitcast(rowb[gs, r, pl.ds(0, HDP)], jnp.int32)
                        ve = plsc.bitcast(jnp.left_shift(v, 16), jnp.float32)
                        vo = plsc.bitcast(v, jnp.float32)
                        acc_e = acc_e + wv * ve
                        acc_o = acc_o + wv * vo
                orow = t + KT * (h // 4)
                ocol = (h % 4) * 32
                outb[os, orow, pl.ds(ocol, 16)] = acc_e
                outb[os, orow, pl.ds(ocol + 16, 16)] = acc_o
            return 0
        lax.fori_loop(0, KT, token_body, 0)

    def put_out(os, ch):
        t0 = tok0 + ch * KT
        pltpu.async_copy(outb.at[os, pl.ds(0, KT)], outlo_ref.at[pl.ds(t0, KT)], osems[os])
        pltpu.async_copy(outb.at[os, pl.ds(KT, KT)], outhi_ref.at[pl.ds(t0, KT)], osems[os])

    stage(0, 0)
    stage(1, 1)
    stage(2, 2)
    wait_meta(0)
    fire(0, 0)

    def quad_body(i, _):
        for k in range(4):
            ch = i * 4 + k
            gs = k % 2
            os = k % 2

            @pl.when(ch + 1 < NCH)
            def _():
                wait_meta((k + 1) % 4)
                fire((k + 1) % 4, (k + 1) % 2)

            drain_gather(gs)

            @pl.when(ch >= 2)
            def _():
                drain_out(os, tok0 + (ch - 2) * KT)

            compute(k, gs, os)
            put_out(os, ch)

            @pl.when(ch + 3 < NCH)
            def _():
                stage((k + 3) % 4, ch + 3)
        return 0

    lax.fori_loop(0, NCH // 4, quad_body, 0)
    drain_out(0, tok0 + (NCH - 2) * KT)
    drain_out(1, tok0 + (NCH - 1) * KT)


@functools.partial(
    pl.kernel,
    out_type=[jax.ShapeDtypeStruct((T, 128), jnp.float32),
              jax.ShapeDtypeStruct((T, 128), jnp.float32)],
    mesh=plsc.VectorSubcoreMesh(core_axis_name="c", subcore_axis_name="s"),
    compiler_params=pltpu.CompilerParams(use_tc_tiling_on_sc=False,
                                         needs_layout_passes=False),
    scratch_types=[
        pltpu.VMEM((4, 2 * KT, 128), jnp.int32),
        pltpu.VMEM((4, 2 * KT, 128), jnp.float32),
        pltpu.VMEM((2, ROWS, HDP), jnp.bfloat16),
        pltpu.VMEM((2, 2 * KT, 128), jnp.float32),
    ] + [pltpu.SemaphoreType.DMA] * 8,
)
def _sc_gather(g_ref, idxlo_ref, idxhi_ref, wgtlo_ref, wgthi_ref,
               outlo_ref, outhi_ref, idxb, wgtb, rowb, outb, *sems):
    _sc_gather_body(g_ref, idxlo_ref, idxhi_ref, wgtlo_ref, wgthi_ref,
                    outlo_ref, outhi_ref, idxb, wgtb, rowb, outb, *sems)


def _pack_body(x_ref, sele_ref, selo_ref, out_ref):
    f32 = jnp.float32
    xb = x_ref[...]
    ev = jnp.dot(xb, sele_ref[...], preferred_element_type=f32).astype(jnp.bfloat16)
    od = jnp.dot(xb, selo_ref[...], preferred_element_type=f32).astype(jnp.bfloat16)
    li = pltpu.bitcast(ev, jnp.uint16).astype(jnp.int32)
    hi = pltpu.bitcast(od, jnp.uint16).astype(jnp.int32) << 16
    out_ref[...] = li | hi


def _proj_body(olo_ref, ohi_ref, wlo_ref, whi_ref, bo_ref, out_ref):
    f32 = jnp.float32
    out_ref[...] = (jnp.dot(olo_ref[...], wlo_ref[...], preferred_element_type=f32)
                    + jnp.dot(ohi_ref[...], whi_ref[...], preferred_element_type=f32)
                    + bo_ref[...])


def kernel(f_query, f_kv, ln_q_g, ln_q_b, ln_kv_g, ln_kv_b, Wq, bq, W1, b1, W2, b2, Wa, ba, Wo, bo):
    f32 = jnp.float32
    X = f_query.reshape(B, C, N).transpose(0, 2, 1).reshape(T, C)
    jq = jnp.arange(NQ) % 32
    W2T3 = W2.T.reshape(C, NH * NP, 3)
    w2x = W2T3[:, jq, 0]; w2y = W2T3[:, jq, 1]; w2z = W2T3[:, jq, 2]
    b23 = b2.reshape(NH * NP, 3)
    b2x = b23[jq, 0][None, :]; b2y = b23[jq, 1][None, :]; b2z = b23[jq, 2][None, :]
    jj = jnp.arange(NH * NP)
    msk = (jj[:, None] // NP == jj[None, :] // NP).astype(f32)
    dup = (jj[:, None] == (jnp.arange(NQ)[None, :] % 32)).astype(f32)

    row = lambda v: v.reshape(1, -1)
    full = lambda s: pl.BlockSpec(s, lambda b, i: tuple(0 for _ in s))
    idxlo, idxhi, wgtlo, wgthi = pl.pallas_call(
        _frontend_body,
        grid=(B, N // TM),
        in_specs=[
            pl.BlockSpec((TM, C), lambda b, i: (b * (N // TM) + i, 0)),
            full((1, C)), full((1, C)),
            full((C, C)), full((1, C)),
            full((C, C)), full((1, C)),
            full((C, NQ)), full((1, NQ)),
            full((C, NQ)), full((1, NQ)),
            full((C, NQ)), full((1, NQ)),
            full((C, NH * NP)), full((1, NH * NP)),
            full((NH * NP, NH * NP)), full((NH * NP, NQ)),
        ],
        out_specs=[pl.BlockSpec((TM, 128), lambda b, i: (b * (N // TM) + i, 0))] * 4,
        out_shape=[jax.ShapeDtypeStruct((T, 128), jnp.int32),
                   jax.ShapeDtypeStruct((T, 128), jnp.int32),
                   jax.ShapeDtypeStruct((T, 128), f32),
                   jax.ShapeDtypeStruct((T, 128), f32)],
    )(X, row(ln_q_g), row(ln_q_b), Wq.T, bq[None, :], W1.T, b1[None, :],
      w2x, b2x, w2y, b2y, w2z, b2z, Wa.T, ba[None, :], msk, dup)

    G = jnp.pad(
        f_kv.reshape(B, C, N).transpose(0, 2, 1).reshape(B, N, NH, HD),
        ((0, 0), (0, 0), (0, 0), (0, HDP - HD)),
    ).astype(jnp.bfloat16).reshape(B * N * NH, HDP)

    O_lo, O_hi = _sc_gather(G, idxlo, idxhi, wgtlo, wgthi)

    perm = jnp.concatenate([jnp.arange(0, HDP, 2), jnp.arange(1, HDP, 2)])
    WoP = jnp.take(
        jnp.pad(Wo.T.reshape(NH, HD, C), ((0, 0), (0, HDP - HD), (0, 0))),
        perm, axis=1).reshape(CP, C)
    w_lo = WoP[:128]
    w_hi = WoP[128:]

    out = pl.pallas_call(
        _proj_body,
        grid=(B, N // TM),
        in_specs=[
            pl.BlockSpec((TM, 128), lambda b, i: (b * (N // TM) + i, 0)),
            pl.BlockSpec((TM, 128), lambda b, i: (b * (N // TM) + i, 0)),
            full((128, C)), full((128, C)), full((1, C)),
        ],
        out_specs=pl.BlockSpec((TM, C), lambda b, i: (b * (N // TM) + i, 0)),
        out_shape=jax.ShapeDtypeStruct((T, C), f32),
    )(O_lo, O_hi, w_lo, w_hi, row(bo))

    return out.reshape(B, N, C).transpose(0, 2, 1).reshape(B, C, H, W, D)
